# baseline (speedup 1.0000x reference)
import functools

import jax
import jax.numpy as jnp
from jax import lax
from jax.experimental import pallas as pl
from jax.experimental.pallas import tpu as pltpu
from jax.experimental.pallas import tpu_sc as plsc

_NUM_WORKERS = 32
_SPLIT = 8
_CHUNKS = (104, 96)


def kernel(indices, embedding_weight):
    batch, tokens = indices.shape
    vocab, dim = embedding_weight.shape
    subdim = dim // _SPLIT
    rows = batch * tokens
    srows = rows * _SPLIT
    rpw = srows // _NUM_WORKERS
    assert sum(_CHUNKS) == rpw

    flat = indices.reshape(-1).astype(jnp.int32)
    idx = flat[:, None] * _SPLIT + jnp.arange(_SPLIT, dtype=jnp.int32)[None, :]
    idx = idx.reshape(_NUM_WORKERS, rpw)
    table = embedding_weight.reshape(vocab * _SPLIT, subdim)

    mesh = plsc.VectorSubcoreMesh(core_axis_name="c", subcore_axis_name="s")

    @functools.partial(
        pl.kernel,
        mesh=mesh,
        out_type=jax.ShapeDtypeStruct((_NUM_WORKERS, rpw, subdim), jnp.float32),
        scratch_types=[
            pltpu.VMEM((rpw,), jnp.int32),
            pltpu.VMEM((rpw, subdim), jnp.float32),
            pltpu.SemaphoreType.DMA,
        ],
    )
    def gather_kernel(table_hbm, idx_hbm, out_hbm, idx_v, rows_v, sem):
        wid = lax.axis_index("s") * 2 + lax.axis_index("c")
        pltpu.sync_copy(idx_hbm.at[wid], idx_v)
        copies = []
        off = 0
        for n in _CHUNKS:
            copies.append(
                pltpu.async_copy(
                    table_hbm.at[idx_v.at[pl.ds(off, n)]],
                    rows_v.at[pl.ds(off, n)],
                    sem,
                )
            )
            off += n
        for c in copies:
            c.wait()
        pltpu.sync_copy(rows_v, out_hbm.at[wid])

    out = gather_kernel(table, idx)
    return out.reshape(batch, tokens, dim)


# speedup vs baseline: 1.3321x; 1.3321x over previous
import functools

import jax
import jax.numpy as jnp
from jax import lax
from jax.experimental import pallas as pl
from jax.experimental.pallas import tpu as pltpu
from jax.experimental.pallas import tpu_sc as plsc

_NUM_WORKERS = 32
_BLOCK = 8


def kernel(indices, embedding_weight):
    batch, tokens = indices.shape
    vocab, dim = embedding_weight.shape
    rows = batch * tokens
    nblocks = rows // _BLOCK
    max_k = -(-nblocks // _NUM_WORKERS)

    idx_flat = indices.reshape(-1).astype(jnp.int32)
    mesh = plsc.VectorSubcoreMesh(core_axis_name="c", subcore_axis_name="s")

    @functools.partial(
        pl.kernel,
        mesh=mesh,
        out_type=jax.ShapeDtypeStruct((rows, dim), jnp.float32),
        scratch_types=[
            pltpu.VMEM((rows,), jnp.int32),
            pltpu.VMEM((_BLOCK, dim), jnp.float32),
            pltpu.VMEM((_BLOCK, dim), jnp.float32),
            pltpu.VMEM((_BLOCK, dim), jnp.float32),
            pltpu.SemaphoreType.DMA,
        ],
    )
    def gather_kernel(table_hbm, idx_hbm, out_hbm, idx_v, b0, b1, b2, sem):
        wid = lax.axis_index("s") * 2 + lax.axis_index("c")
        bufs = (b0, b1, b2)

        pltpu.sync_copy(idx_hbm, idx_v)

        def start_gather(k, buf):
            blk = jnp.minimum(wid + _NUM_WORKERS * k, nblocks - 1)
            return pltpu.async_copy(
                table_hbm.at[idx_v.at[pl.ds(blk * _BLOCK, _BLOCK)]], buf, sem
            )

        def write_back(k, buf):
            blk = wid + _NUM_WORKERS * k
            pltpu.sync_copy(buf, out_hbm.at[pl.ds(blk * _BLOCK, _BLOCK)])

        copies = [start_gather(k, bufs[k]) for k in range(min(3, max_k))]
        for k in range(max_k):
            copies[k].wait()
            if k == 0 and max_k > 3:
                write_back(0, bufs[0])
                copies.append(start_gather(3, bufs[0]))
            elif k < 3:
                full = wid + _NUM_WORKERS * k < nblocks
                if (k + 1) * _NUM_WORKERS <= nblocks:
                    write_back(k, bufs[k])
                else:

                    @pl.when(full)
                    def _():
                        write_back(k, bufs[k])

            else:
                @pl.when(wid + _NUM_WORKERS * k < nblocks)
                def _():
                    write_back(k, bufs[0])

    out = gather_kernel(embedding_weight, idx_flat)
    return out.reshape(batch, tokens, dim)


# speedup vs baseline: 1.6210x; 1.2169x over previous
import functools

import jax
import jax.numpy as jnp
from jax import lax
from jax.experimental import pallas as pl
from jax.experimental.pallas import tpu as pltpu
from jax.experimental.pallas import tpu_sc as plsc

_NUM_WORKERS = 32
_LIGHT = 28


def kernel(indices, embedding_weight):
    batch, tokens = indices.shape
    vocab, dim = embedding_weight.shape
    rows = batch * tokens
    assert _LIGHT * 24 + (_NUM_WORKERS - _LIGHT) * 32 == rows

    idx_flat = indices.reshape(-1).astype(jnp.int32)
    mesh = plsc.VectorSubcoreMesh(core_axis_name="c", subcore_axis_name="s")

    @functools.partial(
        pl.kernel,
        mesh=mesh,
        out_type=jax.ShapeDtypeStruct((rows, dim), jnp.float32),
        scratch_types=[
            pltpu.VMEM((32,), jnp.int32),
            pltpu.VMEM((8, dim), jnp.float32),
            pltpu.VMEM((16, dim), jnp.float32),
            pltpu.SemaphoreType.DMA,
            pltpu.SemaphoreType.DMA,
        ],
    )
    def gather_kernel(table_hbm, idx_hbm, out_hbm, idx_v, buf_a, buf_b, gsem, wsem):
        wid = lax.axis_index("s") * 2 + lax.axis_index("c")
        heavy = wid >= _LIGHT
        off = jnp.where(heavy, _LIGHT * 24 + (wid - _LIGHT) * 32, wid * 24)

        pltpu.sync_copy(idx_hbm.at[pl.ds(off, 32)], idx_v)

        def gather(i0, n, buf):
            return pltpu.async_copy(
                table_hbm.at[idx_v.at[pl.ds(i0, n)]], buf, gsem
            )

        def write(i0, n, buf):
            return pltpu.async_copy(
                buf.at[pl.ds(0, n)] if n != buf.shape[0] else buf,
                out_hbm.at[pl.ds(off + i0, n)],
                wsem,
            )

        g0 = gather(0, 8, buf_a)
        g1 = gather(8, 16, buf_b)
        g0.wait()
        w0 = write(0, 8, buf_a)
        g1.wait()
        w1 = write(8, 16, buf_b)
        w0.wait()

        @pl.when(heavy)
        def _():
            g2 = gather(24, 8, buf_a)
            g2.wait()
            write(24, 8, buf_a).wait()

        w1.wait()

    out = gather_kernel(embedding_weight, idx_flat)
    return out.reshape(batch, tokens, dim)
